# baseline (speedup 1.0000x reference)
import functools

import jax
import jax.numpy as jnp
from jax import lax
from jax.experimental import pallas as pl
from jax.experimental.pallas import tpu as pltpu
from jax.experimental.pallas import tpu_sc as plsc

N = 10000
E = 320000
F = 128
FH = F // 2
G = 64
C = 10

NC = 2
NS = 16
NW = NC * NS

EPW = E // NW
CB0 = 80
NCH0 = EPW // CB0

CB = 128
NCH = 156
TB = (E - NS * NCH * CB) // NS
ACCN = N

NBUF = 6
LOOK = 4

RPT = 624
ZR = 104

NPAD = 10240
HSL = NPAD // NS


def _make_counts():
  mesh = plsc.VectorSubcoreMesh(core_axis_name="c", subcore_axis_name="s")

  @functools.partial(
      pl.kernel,
      out_type=jax.ShapeDtypeStruct((NC, NPAD), jnp.float32),
      mesh=mesh,
      compiler_params=pltpu.CompilerParams(needs_layout_passes=False),
      scratch_types=[
          pltpu.VMEM((NCH0, CB0), jnp.int32),
          pltpu.VMEM((NPAD,), jnp.float32),
          pltpu.VMEM((NS, HSL), jnp.float32),
          pltpu.VMEM((HSL,), jnp.float32),
          pltpu.VMEM_SHARED((NS, NPAD), jnp.float32),
      ],
  )
  def counts(dst_hbm, out_hbm, didx, hist, tbuf, obuf, shared):
    c = lax.axis_index("c")
    s = lax.axis_index("s")
    w = c * NS + s

    def zrow(k, _):
      hist[pl.ds(k * 16, 16)] = jnp.zeros((16,), jnp.float32)
      return 0
    lax.fori_loop(0, NPAD // 16, zrow, 0)

    pltpu.sync_copy(dst_hbm.at[w], didx)

    ones = jnp.ones((16,), jnp.float32)

    def chunk(i, _):
      for j in range(CB0 // 16):
        idx = didx[i, pl.ds(j * 16, 16)]
        plsc.addupdate_scatter(hist, [idx], ones)
      return 0
    lax.fori_loop(0, NCH0, chunk, 0)

    pltpu.sync_copy(hist, shared.at[s])
    plsc.subcore_barrier()
    for t in range(NS):
      pltpu.sync_copy(shared.at[t, pl.ds(s * HSL, HSL)], tbuf.at[t])

    def red(k, _):
      acc = tbuf[0, pl.ds(k * 16, 16)]
      for t in range(1, NS):
        acc = acc + tbuf[t, pl.ds(k * 16, 16)]
      obuf[pl.ds(k * 16, 16)] = acc
      return 0
    lax.fori_loop(0, HSL // 16, red, 0)

    pltpu.sync_copy(obuf, out_hbm.at[c, pl.ds(s * HSL, HSL)])

  return counts


def _make_spmm():
  mesh = plsc.VectorSubcoreMesh(core_axis_name="c", subcore_axis_name="s")

  @functools.partial(
      pl.kernel,
      out_type=jax.ShapeDtypeStruct((N, F), jnp.float32),
      mesh=mesh,
      compiler_params=pltpu.CompilerParams(needs_layout_passes=False,
                                           use_tc_tiling_on_sc=False),
      scratch_types=(
          [
              pltpu.VMEM((NCH, CB), jnp.int32),
              pltpu.VMEM((NCH, CB), jnp.int32),
              pltpu.VMEM((2, TB), jnp.int32),
              pltpu.VMEM((NBUF, CB, FH), jnp.float32),
              pltpu.VMEM_SHARED((ACCN, FH), jnp.float32),
          ]
          + [pltpu.SemaphoreType.DMA] * (2 * NBUF)
      ),
  )
  def spmm(h_hbm, src_hbm, dst_hbm, srct_hbm, dstt_hbm, out_hbm,
           sidx, didx, tidx, rows, acc, *sems):
    gs = sems[:NBUF]
    ss = sems[NBUF:]
    c = lax.axis_index("c")
    s = lax.axis_index("s")

    zb = rows.at[0]

    def zrow(r, _):
      for j in range(FH // 16):
        zb[r, pl.ds(j * 16, 16)] = jnp.zeros((16,), jnp.float32)
      return 0
    lax.fori_loop(0, CB, zrow, 0)
    base = pl.multiple_of(s * RPT, 8)
    for k in range(4):
      pltpu.sync_copy(zb, acc.at[pl.ds(base + k * CB, CB)])
    pltpu.sync_copy(zb.at[pl.ds(0, RPT - 4 * CB)],
                    acc.at[pl.ds(base + 4 * CB, RPT - 4 * CB)])

    @pl.when(s == NS - 1)
    def _():
      pltpu.sync_copy(zb.at[pl.ds(0, N - NS * RPT)],
                      acc.at[pl.ds(NS * RPT, N - NS * RPT)])

    pltpu.sync_copy(src_hbm.at[s], sidx)
    pltpu.sync_copy(dst_hbm.at[s], didx)
    pltpu.sync_copy(srct_hbm.at[s], tidx.at[0])
    pltpu.sync_copy(dstt_hbm.at[s], tidx.at[1])
    plsc.subcore_barrier()

    hsel = h_hbm.at[c]

    def gather(i, b):
      idx = sidx.at[i]
      pltpu.async_copy(hsel.at[idx.at[pl.ds(0, CB // 2)]],
                       rows.at[b].at[pl.ds(0, CB // 2)], gs[b])
      pltpu.async_copy(hsel.at[idx.at[pl.ds(CB // 2, CB // 2)]],
                       rows.at[b].at[pl.ds(CB // 2, CB // 2)], gs[b])

    def wait_gather(b):
      pltpu.make_async_copy(hsel.at[sidx.at[0]], rows.at[b], gs[b]).wait()

    def scatter(i, b):
      pltpu.async_copy(rows.at[b], acc.at[didx.at[i]], ss[b], add=True)

    def wait_scatter(b):
      pltpu.make_async_copy(rows.at[b], acc.at[didx.at[0]], ss[b]).wait()

    for b in range(LOOK):
      gather(b, b)

    def block(q, _):
      for r in range(NBUF):
        i = q * NBUF + r

        @pl.when(i < NCH)
        def _():
          wait_gather(r)
          scatter(i, r)

        nb = (r + LOOK) % NBUF

        @pl.when(i + LOOK < NCH)
        def _():
          @pl.when(i >= NBUF - LOOK)
          def _():
            wait_scatter(nb)
          gather(i + LOOK, nb)
      return 0
    lax.fori_loop(0, (NCH + NBUF - 1) // NBUF, block, 0)

    for b in range(NBUF):
      wait_scatter(b)
    tdst = rows.at[0].at[pl.ds(0, TB)]
    pltpu.async_copy(hsel.at[tidx.at[0]], tdst, gs[0])
    pltpu.make_async_copy(hsel.at[tidx.at[0]], tdst, gs[0]).wait()
    pltpu.sync_copy(tdst, acc.at[tidx.at[1]], add=True)
    plsc.subcore_barrier()

    col = pl.multiple_of(c * FH, 8)
    pltpu.sync_copy(acc.at[pl.ds(base, RPT)],
                    out_hbm.at[pl.ds(base, RPT), pl.ds(col, FH)])

    @pl.when(s == NS - 1)
    def _():
      pltpu.sync_copy(acc.at[pl.ds(NS * RPT, N - NS * RPT)],
                      out_hbm.at[pl.ds(NS * RPT, N - NS * RPT),
                                 pl.ds(col, FH)])

  return spmm


@functools.lru_cache(maxsize=None)
def _counts_k():
  return _make_counts()


@functools.lru_cache(maxsize=None)
def _spmm_k():
  return _make_spmm()


RB = 1000
NBLK = N // RB


def _layer_body(p, c0, c1, h2, bt, wl, wr, bl, out2, pooled):
  i = pl.program_id(0)
  cnt = c0[...] + c1[...]
  inv = 1.0 / jnp.maximum(cnt, 1.0)
  mean = p[...] * inv
  wr_ = wr[...]
  hn = (jnp.dot(mean, wl[...], preferred_element_type=jnp.float32)
        + jnp.dot(h2[0], wr_[:FH, :], preferred_element_type=jnp.float32)
        + jnp.dot(h2[1], wr_[FH:, :], preferred_element_type=jnp.float32)
        + bl[...])
  out2[0] = hn[:, :FH]
  out2[1] = hn[:, FH:]

  @pl.when(i == 0)
  def _():
    pooled[...] = jnp.full((G, F), -jnp.inf, jnp.float32)

  gfirst = bt[0, 0]
  glast = bt[RB - 1, 0]

  def gbody(g, _):
    m = jnp.where(bt[...] == g, 0.0, -jnp.inf)
    v = jnp.max(hn + m, axis=0, keepdims=True)
    pooled[pl.ds(g, 1), :] = jnp.maximum(pooled[pl.ds(g, 1), :], v)
    return 0
  lax.fori_loop(gfirst, glast + 1, gbody, 0)


def _tc_layer(p, c0, c1, h2, bt, wl, wr, bl):
  return pl.pallas_call(
      _layer_body,
      grid=(NBLK,),
      in_specs=[
          pl.BlockSpec((RB, F), lambda i: (i, 0)),
          pl.BlockSpec((RB, 1), lambda i: (i, 0)),
          pl.BlockSpec((RB, 1), lambda i: (i, 0)),
          pl.BlockSpec((2, RB, FH), lambda i: (0, i, 0)),
          pl.BlockSpec((RB, 1), lambda i: (i, 0)),
          pl.BlockSpec((F, F), lambda i: (0, 0)),
          pl.BlockSpec((F, F), lambda i: (0, 0)),
          pl.BlockSpec((1, F), lambda i: (0, 0)),
      ],
      out_specs=[
          pl.BlockSpec((2, RB, FH), lambda i: (0, i, 0)),
          pl.BlockSpec((G, F), lambda i: (0, 0)),
      ],
      out_shape=[
          jax.ShapeDtypeStruct((2, N, FH), jnp.float32),
          jax.ShapeDtypeStruct((G, F), jnp.float32),
      ],
  )(p, c0, c1, h2, bt, wl, wr, bl)


def _last_body(p, c0, c1, h2, bt, wl, wr, bl, q0, q1, w1, b1, w2, b2,
               out, pooled):
  i = pl.program_id(0)
  cnt = c0[...] + c1[...]
  inv = 1.0 / jnp.maximum(cnt, 1.0)
  mean = p[...] * inv
  wr_ = wr[...]
  hn = (jnp.dot(mean, wl[...], preferred_element_type=jnp.float32)
        + jnp.dot(h2[0], wr_[:FH, :], preferred_element_type=jnp.float32)
        + jnp.dot(h2[1], wr_[FH:, :], preferred_element_type=jnp.float32)
        + bl[...])

  @pl.when(i == 0)
  def _():
    pooled[...] = jnp.full((G, F), -jnp.inf, jnp.float32)

  gfirst = bt[0, 0]
  glast = bt[RB - 1, 0]

  def gbody(g, _):
    m = jnp.where(bt[...] == g, 0.0, -jnp.inf)
    v = jnp.max(hn + m, axis=0, keepdims=True)
    pooled[pl.ds(g, 1), :] = jnp.maximum(pooled[pl.ds(g, 1), :], v)
    return 0
  lax.fori_loop(gfirst, glast + 1, gbody, 0)

  @pl.when(i == NBLK - 1)
  def _():
    hcat = jnp.concatenate([q0[...], q1[...], pooled[...]], axis=1)
    z = jnp.maximum(
        jnp.dot(hcat, w1[...], preferred_element_type=jnp.float32) + b1[...],
        0.0)
    out[...] = jnp.dot(z, w2[...], preferred_element_type=jnp.float32) + b2[...]


def _tc_last(p, c0, c1, h2, bt, wl, wr, bl, q0, q1, w1, b1, w2, b2):
  return pl.pallas_call(
      _last_body,
      grid=(NBLK,),
      in_specs=[
          pl.BlockSpec((RB, F), lambda i: (i, 0)),
          pl.BlockSpec((RB, 1), lambda i: (i, 0)),
          pl.BlockSpec((RB, 1), lambda i: (i, 0)),
          pl.BlockSpec((2, RB, FH), lambda i: (0, i, 0)),
          pl.BlockSpec((RB, 1), lambda i: (i, 0)),
          pl.BlockSpec((F, F), lambda i: (0, 0)),
          pl.BlockSpec((F, F), lambda i: (0, 0)),
          pl.BlockSpec((1, F), lambda i: (0, 0)),
          pl.BlockSpec((G, F), lambda i: (0, 0)),
          pl.BlockSpec((G, F), lambda i: (0, 0)),
          pl.BlockSpec((3 * F, F), lambda i: (0, 0)),
          pl.BlockSpec((1, F), lambda i: (0, 0)),
          pl.BlockSpec((F, C), lambda i: (0, 0)),
          pl.BlockSpec((1, C), lambda i: (0, 0)),
      ],
      out_specs=pl.BlockSpec((G, C), lambda i: (0, 0)),
      out_shape=jax.ShapeDtypeStruct((G, C), jnp.float32),
      scratch_shapes=[pltpu.VMEM((G, F), jnp.float32)],
  )(p, c0, c1, h2, bt, wl, wr, bl, q0, q1, w1, b1, w2, b2)


def kernel(x, edge_index, batch, Wl0, bl0, Wr0, Wl1, bl1, Wr1, Wl2, bl2, Wr2,
           fc1_W, fc1_b, fc2_W, fc2_b):
  src_c = edge_index[0].reshape(NW, NCH0, CB0)
  dst_c = edge_index[1].reshape(NW, NCH0, CB0)

  nm = NS * NCH * CB
  src_s = edge_index[0][:nm].reshape(NS, NCH, CB)
  dst_s = edge_index[1][:nm].reshape(NS, NCH, CB)
  src_t = edge_index[0][nm:].reshape(NS, TB)
  dst_t = edge_index[1][nm:].reshape(NS, TB)

  cnt = _counts_k()(dst_c)
  c0 = cnt[0, :N].reshape(N, 1)
  c1 = cnt[1, :N].reshape(N, 1)
  bt = batch.reshape(N, 1)

  h2 = jnp.stack([x[:, :FH], x[:, FH:]])
  pooled = []
  for wl, bl, wr in ((Wl0, bl0, Wr0), (Wl1, bl1, Wr1)):
    p = _spmm_k()(h2, src_s, dst_s, src_t, dst_t)
    h2, pool_l = _tc_layer(p, c0, c1, h2, bt, wl, wr, bl.reshape(1, F))
    pooled.append(pool_l)

  p = _spmm_k()(h2, src_s, dst_s, src_t, dst_t)
  return _tc_last(p, c0, c1, h2, bt, Wl2, Wr2, bl2.reshape(1, F),
                  pooled[0], pooled[1], fc1_W, fc1_b.reshape(1, F),
                  fc2_W, fc2_b.reshape(1, C))


# speedup vs baseline: 1.0035x; 1.0035x over previous
import functools

import jax
import jax.numpy as jnp
from jax import lax
from jax.experimental import pallas as pl
from jax.experimental.pallas import tpu as pltpu
from jax.experimental.pallas import tpu_sc as plsc

N = 10000
E = 320000
F = 128
FH = F // 2
G = 64
C = 10

NC = 2
NS = 16
NW = NC * NS

EPW = E // NW
CB0 = 80
NCH0 = EPW // CB0

CB = 128
NCH = 156
TB = (E - NS * NCH * CB) // NS
ACCN = N

NBUF = 6
LOOK = 4

RPT = 624
ZR = 104

NPAD = 10240
HSL = NPAD // NS


def _make_counts():
  mesh = plsc.VectorSubcoreMesh(core_axis_name="c", subcore_axis_name="s")

  @functools.partial(
      pl.kernel,
      out_type=jax.ShapeDtypeStruct((NC, NPAD), jnp.float32),
      mesh=mesh,
      compiler_params=pltpu.CompilerParams(needs_layout_passes=False),
      scratch_types=[
          pltpu.VMEM((NCH0, CB0), jnp.int32),
          pltpu.VMEM((NPAD,), jnp.float32),
          pltpu.VMEM((NS, HSL), jnp.float32),
          pltpu.VMEM((HSL,), jnp.float32),
          pltpu.VMEM_SHARED((NS, NPAD), jnp.float32),
      ],
  )
  def counts(dst_hbm, out_hbm, didx, hist, tbuf, obuf, shared):
    c = lax.axis_index("c")
    s = lax.axis_index("s")
    w = c * NS + s

    def zrow(k, _):
      hist[pl.ds(k * 16, 16)] = jnp.zeros((16,), jnp.float32)
      return 0
    lax.fori_loop(0, NPAD // 16, zrow, 0)

    pltpu.sync_copy(dst_hbm.at[w], didx)

    ones = jnp.ones((16,), jnp.float32)

    def chunk(i, _):
      for j in range(CB0 // 16):
        idx = didx[i, pl.ds(j * 16, 16)]
        plsc.addupdate_scatter(hist, [idx], ones)
      return 0
    lax.fori_loop(0, NCH0, chunk, 0)

    pltpu.sync_copy(hist, shared.at[s])
    plsc.subcore_barrier()
    for t in range(NS):
      pltpu.sync_copy(shared.at[t, pl.ds(s * HSL, HSL)], tbuf.at[t])

    def red(k, _):
      acc = tbuf[0, pl.ds(k * 16, 16)]
      for t in range(1, NS):
        acc = acc + tbuf[t, pl.ds(k * 16, 16)]
      obuf[pl.ds(k * 16, 16)] = acc
      return 0
    lax.fori_loop(0, HSL // 16, red, 0)

    pltpu.sync_copy(obuf, out_hbm.at[c, pl.ds(s * HSL, HSL)])

  return counts


def _make_spmm():
  mesh = plsc.VectorSubcoreMesh(core_axis_name="c", subcore_axis_name="s")

  @functools.partial(
      pl.kernel,
      out_type=jax.ShapeDtypeStruct((N, F), jnp.float32),
      mesh=mesh,
      compiler_params=pltpu.CompilerParams(needs_layout_passes=False,
                                           use_tc_tiling_on_sc=False),
      scratch_types=(
          [
              pltpu.VMEM((NCH, CB), jnp.int32),
              pltpu.VMEM((NCH, CB), jnp.int32),
              pltpu.VMEM((2, TB), jnp.int32),
              pltpu.VMEM((NBUF, CB, FH), jnp.float32),
              pltpu.VMEM_SHARED((ACCN, FH), jnp.float32),
          ]
          + [pltpu.SemaphoreType.DMA] * (2 * NBUF)
      ),
  )
  def spmm(h_hbm, src_hbm, dst_hbm, srct_hbm, dstt_hbm, out_hbm,
           sidx, didx, tidx, rows, acc, *sems):
    gs = sems[:NBUF]
    ss = sems[NBUF:]
    c = lax.axis_index("c")
    s = lax.axis_index("s")

    zb = rows.at[0]

    def zrow(r, _):
      for j in range(FH // 16):
        zb[r, pl.ds(j * 16, 16)] = jnp.zeros((16,), jnp.float32)
      return 0
    lax.fori_loop(0, CB, zrow, 0)
    base = pl.multiple_of(s * RPT, 8)
    for k in range(4):
      pltpu.sync_copy(zb, acc.at[pl.ds(base + k * CB, CB)])
    pltpu.sync_copy(zb.at[pl.ds(0, RPT - 4 * CB)],
                    acc.at[pl.ds(base + 4 * CB, RPT - 4 * CB)])

    @pl.when(s == NS - 1)
    def _():
      pltpu.sync_copy(zb.at[pl.ds(0, N - NS * RPT)],
                      acc.at[pl.ds(NS * RPT, N - NS * RPT)])

    pltpu.sync_copy(src_hbm.at[s], sidx)
    pltpu.sync_copy(dst_hbm.at[s], didx)
    pltpu.sync_copy(srct_hbm.at[s], tidx.at[0])
    pltpu.sync_copy(dstt_hbm.at[s], tidx.at[1])
    plsc.subcore_barrier()

    hsel = h_hbm.at[c]

    def gather(i, b):
      pltpu.async_copy(hsel.at[sidx.at[i]], rows.at[b], gs[b])

    def wait_gather(b):
      pltpu.make_async_copy(hsel.at[sidx.at[0]], rows.at[b], gs[b]).wait()

    def scatter(i, b):
      pltpu.async_copy(rows.at[b], acc.at[didx.at[i]], ss[b], add=True)

    def wait_scatter(b):
      pltpu.make_async_copy(rows.at[b], acc.at[didx.at[0]], ss[b]).wait()

    for b in range(LOOK):
      gather(b, b)

    def block(q, _):
      for r in range(NBUF):
        i = q * NBUF + r

        @pl.when(i < NCH)
        def _():
          wait_gather(r)
          scatter(i, r)

        nb = (r + LOOK) % NBUF

        @pl.when(i + LOOK < NCH)
        def _():
          @pl.when(i >= NBUF - LOOK)
          def _():
            wait_scatter(nb)
          gather(i + LOOK, nb)
      return 0
    lax.fori_loop(0, (NCH + NBUF - 1) // NBUF, block, 0)

    for b in range(NBUF):
      wait_scatter(b)
    tdst = rows.at[0].at[pl.ds(0, TB)]
    pltpu.async_copy(hsel.at[tidx.at[0]], tdst, gs[0])
    pltpu.make_async_copy(hsel.at[tidx.at[0]], tdst, gs[0]).wait()
    pltpu.sync_copy(tdst, acc.at[tidx.at[1]], add=True)
    plsc.subcore_barrier()

    col = pl.multiple_of(c * FH, 8)
    pltpu.sync_copy(acc.at[pl.ds(base, RPT)],
                    out_hbm.at[pl.ds(base, RPT), pl.ds(col, FH)])

    @pl.when(s == NS - 1)
    def _():
      pltpu.sync_copy(acc.at[pl.ds(NS * RPT, N - NS * RPT)],
                      out_hbm.at[pl.ds(NS * RPT, N - NS * RPT),
                                 pl.ds(col, FH)])

  return spmm


@functools.lru_cache(maxsize=None)
def _counts_k():
  return _make_counts()


@functools.lru_cache(maxsize=None)
def _spmm_k():
  return _make_spmm()


RB = 1000
NBLK = N // RB


def _layer_body(p, c0, c1, h2, bt, wl, wr, bl, out2, pooled):
  i = pl.program_id(0)
  cnt = c0[...] + c1[...]
  inv = 1.0 / jnp.maximum(cnt, 1.0)
  mean = p[...] * inv
  wr_ = wr[...]
  hn = (jnp.dot(mean, wl[...], preferred_element_type=jnp.float32)
        + jnp.dot(h2[0], wr_[:FH, :], preferred_element_type=jnp.float32)
        + jnp.dot(h2[1], wr_[FH:, :], preferred_element_type=jnp.float32)
        + bl[...])
  out2[0] = hn[:, :FH]
  out2[1] = hn[:, FH:]

  @pl.when(i == 0)
  def _():
    pooled[...] = jnp.full((G, F), -jnp.inf, jnp.float32)

  gfirst = bt[0, 0]
  glast = bt[RB - 1, 0]

  def gbody(g, _):
    m = jnp.where(bt[...] == g, 0.0, -jnp.inf)
    v = jnp.max(hn + m, axis=0, keepdims=True)
    pooled[pl.ds(g, 1), :] = jnp.maximum(pooled[pl.ds(g, 1), :], v)
    return 0
  lax.fori_loop(gfirst, glast + 1, gbody, 0)


def _tc_layer(p, c0, c1, h2, bt, wl, wr, bl):
  return pl.pallas_call(
      _layer_body,
      grid=(NBLK,),
      in_specs=[
          pl.BlockSpec((RB, F), lambda i: (i, 0)),
          pl.BlockSpec((RB, 1), lambda i: (i, 0)),
          pl.BlockSpec((RB, 1), lambda i: (i, 0)),
          pl.BlockSpec((2, RB, FH), lambda i: (0, i, 0)),
          pl.BlockSpec((RB, 1), lambda i: (i, 0)),
          pl.BlockSpec((F, F), lambda i: (0, 0)),
          pl.BlockSpec((F, F), lambda i: (0, 0)),
          pl.BlockSpec((1, F), lambda i: (0, 0)),
      ],
      out_specs=[
          pl.BlockSpec((2, RB, FH), lambda i: (0, i, 0)),
          pl.BlockSpec((G, F), lambda i: (0, 0)),
      ],
      out_shape=[
          jax.ShapeDtypeStruct((2, N, FH), jnp.float32),
          jax.ShapeDtypeStruct((G, F), jnp.float32),
      ],
  )(p, c0, c1, h2, bt, wl, wr, bl)


def _last_body(p, c0, c1, h2, bt, wl, wr, bl, q0, q1, w1, b1, w2, b2,
               out, pooled):
  i = pl.program_id(0)
  cnt = c0[...] + c1[...]
  inv = 1.0 / jnp.maximum(cnt, 1.0)
  mean = p[...] * inv
  wr_ = wr[...]
  hn = (jnp.dot(mean, wl[...], preferred_element_type=jnp.float32)
        + jnp.dot(h2[0], wr_[:FH, :], preferred_element_type=jnp.float32)
        + jnp.dot(h2[1], wr_[FH:, :], preferred_element_type=jnp.float32)
        + bl[...])

  @pl.when(i == 0)
  def _():
    pooled[...] = jnp.full((G, F), -jnp.inf, jnp.float32)

  gfirst = bt[0, 0]
  glast = bt[RB - 1, 0]

  def gbody(g, _):
    m = jnp.where(bt[...] == g, 0.0, -jnp.inf)
    v = jnp.max(hn + m, axis=0, keepdims=True)
    pooled[pl.ds(g, 1), :] = jnp.maximum(pooled[pl.ds(g, 1), :], v)
    return 0
  lax.fori_loop(gfirst, glast + 1, gbody, 0)

  @pl.when(i == NBLK - 1)
  def _():
    hcat = jnp.concatenate([q0[...], q1[...], pooled[...]], axis=1)
    z = jnp.maximum(
        jnp.dot(hcat, w1[...], preferred_element_type=jnp.float32) + b1[...],
        0.0)
    out[...] = jnp.dot(z, w2[...], preferred_element_type=jnp.float32) + b2[...]


def _tc_last(p, c0, c1, h2, bt, wl, wr, bl, q0, q1, w1, b1, w2, b2):
  return pl.pallas_call(
      _last_body,
      grid=(NBLK,),
      in_specs=[
          pl.BlockSpec((RB, F), lambda i: (i, 0)),
          pl.BlockSpec((RB, 1), lambda i: (i, 0)),
          pl.BlockSpec((RB, 1), lambda i: (i, 0)),
          pl.BlockSpec((2, RB, FH), lambda i: (0, i, 0)),
          pl.BlockSpec((RB, 1), lambda i: (i, 0)),
          pl.BlockSpec((F, F), lambda i: (0, 0)),
          pl.BlockSpec((F, F), lambda i: (0, 0)),
          pl.BlockSpec((1, F), lambda i: (0, 0)),
          pl.BlockSpec((G, F), lambda i: (0, 0)),
          pl.BlockSpec((G, F), lambda i: (0, 0)),
          pl.BlockSpec((3 * F, F), lambda i: (0, 0)),
          pl.BlockSpec((1, F), lambda i: (0, 0)),
          pl.BlockSpec((F, C), lambda i: (0, 0)),
          pl.BlockSpec((1, C), lambda i: (0, 0)),
      ],
      out_specs=pl.BlockSpec((G, C), lambda i: (0, 0)),
      out_shape=jax.ShapeDtypeStruct((G, C), jnp.float32),
      scratch_shapes=[pltpu.VMEM((G, F), jnp.float32)],
  )(p, c0, c1, h2, bt, wl, wr, bl, q0, q1, w1, b1, w2, b2)


def kernel(x, edge_index, batch, Wl0, bl0, Wr0, Wl1, bl1, Wr1, Wl2, bl2, Wr2,
           fc1_W, fc1_b, fc2_W, fc2_b):
  src_c = edge_index[0].reshape(NW, NCH0, CB0)
  dst_c = edge_index[1].reshape(NW, NCH0, CB0)

  nm = NS * NCH * CB
  src_s = edge_index[0][:nm].reshape(NS, NCH, CB)
  dst_s = edge_index[1][:nm].reshape(NS, NCH, CB)
  src_t = edge_index[0][nm:].reshape(NS, TB)
  dst_t = edge_index[1][nm:].reshape(NS, TB)

  cnt = _counts_k()(dst_c)
  c0 = cnt[0, :N].reshape(N, 1)
  c1 = cnt[1, :N].reshape(N, 1)
  bt = batch.reshape(N, 1)

  h2 = jnp.stack([x[:, :FH], x[:, FH:]])
  pooled = []
  for wl, bl, wr in ((Wl0, bl0, Wr0), (Wl1, bl1, Wr1)):
    p = _spmm_k()(h2, src_s, dst_s, src_t, dst_t)
    h2, pool_l = _tc_layer(p, c0, c1, h2, bt, wl, wr, bl.reshape(1, F))
    pooled.append(pool_l)

  p = _spmm_k()(h2, src_s, dst_s, src_t, dst_t)
  return _tc_last(p, c0, c1, h2, bt, Wl2, Wr2, bl2.reshape(1, F),
                  pooled[0], pooled[1], fc1_W, fc1_b.reshape(1, F),
                  fc2_W, fc2_b.reshape(1, C))
